# Initial kernel scaffold; baseline (speedup 1.0000x reference)
import jax
import jax.numpy as jnp
from jax.experimental import pallas as pl


def kernel(queries, keys, values):
    raise NotImplementedError("write your pallas kernel here")


# speedup vs baseline: 3.0118x; 3.0118x over previous
import math

import jax
import jax.numpy as jnp
import numpy as np
from jax.experimental import pallas as pl
from jax.experimental.pallas import tpu as pltpu

TIME_LEN = 64
N_WT = 16
FACTOR = 2
B, L, D = 8, TIME_LEN * N_WT, 256
U = int(np.ceil(FACTOR * np.sqrt(L)))


def _threefry2x32(k0, k1, count):
    def rotl(x, r):
        return ((x << np.uint32(r)) | (x >> np.uint32(32 - r))).astype(np.uint32)

    ks = [np.uint32(k0), np.uint32(k1),
          np.uint32(np.uint32(k0) ^ np.uint32(k1) ^ np.uint32(0x1BD11BDA))]
    rot = [13, 15, 26, 6, 17, 29, 16, 24]
    n = count.size // 2
    x0 = (count[:n] + ks[0]).astype(np.uint32)
    x1 = (count[n:] + ks[1]).astype(np.uint32)
    for i in range(5):
        for r in rot[:4] if i % 2 == 0 else rot[4:]:
            x0 = (x0 + x1).astype(np.uint32)
            x1 = (rotl(x1, r) ^ x0).astype(np.uint32)
        x0 = (x0 + ks[(i + 1) % 3]).astype(np.uint32)
        x1 = (x1 + ks[(i + 2) % 3] + np.uint32(i + 1)).astype(np.uint32)
    return x0, x1


def _sample_indices():
    s0, s1 = _threefry2x32(0, 42, np.array([0, 0, 0, 0, 0, 1, 2, 3], np.uint32))
    n = L * U
    i = np.arange(n, dtype=np.uint64)
    hi = (i >> np.uint64(32)).astype(np.uint32)
    lo = (i & np.uint64(0xFFFFFFFF)).astype(np.uint32)
    b0, b1 = _threefry2x32(s0[1], s1[1], np.concatenate([hi, lo]))
    return ((b0 ^ b1) % np.uint32(L)).astype(np.int32).reshape(L, U)


_IDX = _sample_indices()
_CNT = np.zeros((L, L), np.float32)
np.add.at(_CNT, (np.repeat(np.arange(L), U), _IDX.ravel()), 1.0)

_NEG = np.float32(-1e30)


def _fused_kernel(q_ref, k_ref, v_ref, cnt_ref, out_ref):
    q = q_ref[0]
    k = k_ref[0]
    v = v_ref[0]
    cnt = cnt_ref[...]
    scale = 1.0 / math.sqrt(D)

    qk = jax.lax.dot_general(q, k, (((1,), (1,)), ((), ())),
                             preferred_element_type=jnp.float32)
    hit = cnt > 0.0
    mx = jnp.max(jnp.where(hit, qk, _NEG), axis=1)
    sm = jnp.sum(qk * cnt, axis=1)
    m = (mx - sm * (1.0 / L)).reshape(1, L)

    lane = jax.lax.broadcasted_iota(jnp.int32, (1, L), 1)
    vv, idx = m, lane
    kk = 2
    while kk <= L:
        j = kk // 2
        while j > 0:
            left = (lane & j) == 0
            pv = jnp.where(left, pltpu.roll(vv, L - j, 1), pltpu.roll(vv, j, 1))
            pidx = jnp.where(left, pltpu.roll(idx, L - j, 1), pltpu.roll(idx, j, 1))
            wins = (vv > pv) | ((vv == pv) & (idx < pidx))
            keep = left == (((lane & kk) == 0) == wins)
            vv = jnp.where(keep, vv, pv)
            idx = jnp.where(keep, idx, pidx)
            j //= 2
        kk *= 2
    top = idx[:, :U]

    subl = jax.lax.broadcasted_iota(jnp.int32, (L, U), 0)
    pt = (subl == jnp.broadcast_to(top, (L, U))).astype(jnp.float32)
    sel = jnp.max(pt, axis=1, keepdims=True)

    scores = jax.lax.dot_general(pt, qk, (((0,), (0,)), ((), ())),
                                 preferred_element_type=jnp.float32) * scale
    lblk = (jax.lax.broadcasted_iota(jnp.int32, (L, 1), 0) // TIME_LEN)
    qblk = jax.lax.dot_general(pt, lblk.astype(jnp.float32),
                               (((0,), (0,)), ((), ())),
                               preferred_element_type=jnp.float32)
    kb = jax.lax.broadcasted_iota(jnp.int32, (U, L), 1) // TIME_LEN
    allowed = kb <= qblk.astype(jnp.int32)
    scores = jnp.where(allowed, scores, _NEG)
    smax = jnp.max(scores, axis=1, keepdims=True)
    e = jnp.exp(scores - smax)
    attn = e / jnp.sum(e, axis=1, keepdims=True)
    out64 = jax.lax.dot_general(attn, v, (((1,), (0,)), ((), ())),
                                preferred_element_type=jnp.float32)

    row = jax.lax.broadcasted_iota(jnp.int32, (128, 128), 0)
    col = jax.lax.broadcasted_iota(jnp.int32, (128, 128), 1)
    tril = (col <= row).astype(jnp.float32)
    carry = jnp.zeros((1, D), jnp.float32)
    blocks = []
    for t in range(L // 128):
        cs = jax.lax.dot_general(tril, v[t * 128:(t + 1) * 128, :],
                                 (((1,), (0,)), ((), ())),
                                 preferred_element_type=jnp.float32) + carry
        blocks.append(cs)
        carry = cs[127:128, :]
    ctx = jnp.concatenate(blocks, axis=0)

    out_ref[0] = ctx * (1.0 - sel) + jax.lax.dot_general(
        pt, out64, (((1,), (0,)), ((), ())),
        preferred_element_type=jnp.float32)


def kernel(queries, keys, values):
    cnt = jnp.asarray(_CNT)
    return pl.pallas_call(
        _fused_kernel,
        grid=(B,),
        in_specs=[
            pl.BlockSpec((1, L, D), lambda b: (b, 0, 0)),
            pl.BlockSpec((1, L, D), lambda b: (b, 0, 0)),
            pl.BlockSpec((1, L, D), lambda b: (b, 0, 0)),
            pl.BlockSpec((L, L), lambda b: (0, 0)),
        ],
        out_specs=pl.BlockSpec((1, L, D), lambda b: (b, 0, 0)),
        out_shape=jax.ShapeDtypeStruct((B, L, D), jnp.float32),
    )(queries, keys, values, cnt)


# speedup vs baseline: 4.0188x; 1.3343x over previous
import math

import jax
import jax.numpy as jnp
import numpy as np
from jax.experimental import pallas as pl
from jax.experimental.pallas import tpu as pltpu

TIME_LEN = 64
N_WT = 16
FACTOR = 2
B, L, D = 8, TIME_LEN * N_WT, 256
U = int(np.ceil(FACTOR * np.sqrt(L)))


def _threefry2x32(k0, k1, count):
    def rotl(x, r):
        return ((x << np.uint32(r)) | (x >> np.uint32(32 - r))).astype(np.uint32)

    ks = [np.uint32(k0), np.uint32(k1),
          np.uint32(np.uint32(k0) ^ np.uint32(k1) ^ np.uint32(0x1BD11BDA))]
    rot = [13, 15, 26, 6, 17, 29, 16, 24]
    n = count.size // 2
    x0 = (count[:n] + ks[0]).astype(np.uint32)
    x1 = (count[n:] + ks[1]).astype(np.uint32)
    for i in range(5):
        for r in rot[:4] if i % 2 == 0 else rot[4:]:
            x0 = (x0 + x1).astype(np.uint32)
            x1 = (rotl(x1, r) ^ x0).astype(np.uint32)
        x0 = (x0 + ks[(i + 1) % 3]).astype(np.uint32)
        x1 = (x1 + ks[(i + 2) % 3] + np.uint32(i + 1)).astype(np.uint32)
    return x0, x1


def _sample_indices():
    s0, s1 = _threefry2x32(0, 42, np.array([0, 0, 0, 0, 0, 1, 2, 3], np.uint32))
    n = L * U
    i = np.arange(n, dtype=np.uint64)
    hi = (i >> np.uint64(32)).astype(np.uint32)
    lo = (i & np.uint64(0xFFFFFFFF)).astype(np.uint32)
    b0, b1 = _threefry2x32(s0[1], s1[1], np.concatenate([hi, lo]))
    return ((b0 ^ b1) % np.uint32(L)).astype(np.int32).reshape(L, U)


_IDX = _sample_indices()
_CNT = np.zeros((L, L), np.float32)
np.add.at(_CNT, (np.repeat(np.arange(L), U), _IDX.ravel()), 1.0)

_NEG = np.float32(-1e30)


_SUB = L // 128


def _qk_phase(q, k, cnt, qk_ref, m_ref):
    qk = jax.lax.dot_general(q, k, (((1,), (1,)), ((), ())),
                             preferred_element_type=jnp.float32)
    qk_ref[...] = qk
    hit = cnt > 0.0
    mx = jnp.max(jnp.where(hit, qk, _NEG), axis=1)
    sm = jnp.sum(qk * cnt, axis=1)
    m_ref[...] = (mx - sm * (1.0 / L)).reshape(_SUB, 128)


def _attn_phase(qk_ref, m_ref, v, out_ref):
    scale = 1.0 / math.sqrt(D)

    s_io = jax.lax.broadcasted_iota(jnp.int32, (_SUB, 128), 0)
    c_io = jax.lax.broadcasted_iota(jnp.int32, (_SUB, 128), 1)
    e = s_io * 128 + c_io
    vv = m_ref[...]
    idx = e
    kk = 2
    while kk <= L:
        j = kk // 2
        while j > 0:
            left = (e & j) == 0
            if j < 128:
                ax, amt, n = 1, j, 128
            else:
                ax, amt, n = 0, j // 128, _SUB
            pv = jnp.where(left, pltpu.roll(vv, n - amt, ax),
                           pltpu.roll(vv, amt, ax))
            pidx = jnp.where(left, pltpu.roll(idx, n - amt, ax),
                             pltpu.roll(idx, amt, ax))
            wins = (vv > pv) | ((vv == pv) & (idx < pidx))
            keep = left == (((e & kk) == 0) == wins)
            vv = jnp.where(keep, vv, pv)
            idx = jnp.where(keep, idx, pidx)
            j //= 2
        kk *= 2
    top = idx[0:1, :U]

    subl = jax.lax.broadcasted_iota(jnp.int32, (L, U), 0)
    pt = (subl == jnp.broadcast_to(top, (L, U))).astype(jnp.float32)
    sel = jnp.max(pt, axis=1, keepdims=True)

    scores = jax.lax.dot_general(pt, qk_ref[...], (((0,), (0,)), ((), ())),
                                 preferred_element_type=jnp.float32) * scale
    lblk = jax.lax.broadcasted_iota(jnp.int32, (L, 1), 0) // TIME_LEN
    qblk = jax.lax.dot_general(pt, lblk.astype(jnp.float32),
                               (((0,), (0,)), ((), ())),
                               preferred_element_type=jnp.float32)
    kb = jax.lax.broadcasted_iota(jnp.int32, (U, L), 1) // TIME_LEN
    allowed = kb <= qblk.astype(jnp.int32)
    scores = jnp.where(allowed, scores, _NEG)
    smax = jnp.max(scores, axis=1, keepdims=True)
    ex = jnp.exp(scores - smax)
    attn = ex / jnp.sum(ex, axis=1, keepdims=True)
    out64 = jax.lax.dot_general(attn, v, (((1,), (0,)), ((), ())),
                                preferred_element_type=jnp.float32)

    row = jax.lax.broadcasted_iota(jnp.int32, (128, 128), 0)
    col = jax.lax.broadcasted_iota(jnp.int32, (128, 128), 1)
    tril = (col <= row).astype(jnp.float32)
    carry = jnp.zeros((1, D), jnp.float32)
    blocks = []
    for t in range(L // 128):
        cs = jax.lax.dot_general(tril, v[t * 128:(t + 1) * 128, :],
                                 (((1,), (0,)), ((), ())),
                                 preferred_element_type=jnp.float32) + carry
        blocks.append(cs)
        carry = cs[127:128, :]
    ctx = jnp.concatenate(blocks, axis=0)

    out_ref[0] = ctx * (1.0 - sel) + jax.lax.dot_general(
        pt, out64, (((1,), (0,)), ((), ())),
        preferred_element_type=jnp.float32)


def _pipelined_kernel(q_ref, k_ref, v_ref, cnt_ref, out_ref,
                      qk0, qk1, m0, m1):
    b = pl.program_id(0)
    even = b % 2 == 0

    @pl.when((b < B) & even)
    def _qk_even():
        _qk_phase(q_ref[0], k_ref[0], cnt_ref[...], qk0, m0)

    @pl.when((b < B) & jnp.logical_not(even))
    def _qk_odd():
        _qk_phase(q_ref[0], k_ref[0], cnt_ref[...], qk1, m1)

    @pl.when((b > 0) & jnp.logical_not(even))
    def _attn_even():
        _attn_phase(qk0, m0, v_ref[0], out_ref)

    @pl.when((b > 0) & even)
    def _attn_odd():
        _attn_phase(qk1, m1, v_ref[0], out_ref)


def kernel(queries, keys, values):
    cnt = jnp.asarray(_CNT)
    return pl.pallas_call(
        _pipelined_kernel,
        grid=(B + 1,),
        in_specs=[
            pl.BlockSpec((1, L, D), lambda b: (jnp.minimum(b, B - 1), 0, 0)),
            pl.BlockSpec((1, L, D), lambda b: (jnp.minimum(b, B - 1), 0, 0)),
            pl.BlockSpec((1, L, D), lambda b: (jnp.maximum(b - 1, 0), 0, 0)),
            pl.BlockSpec((L, L), lambda b: (0, 0)),
        ],
        out_specs=pl.BlockSpec((1, L, D), lambda b: (jnp.maximum(b - 1, 0), 0, 0)),
        out_shape=jax.ShapeDtypeStruct((B, L, D), jnp.float32),
        scratch_shapes=[
            pltpu.VMEM((L, L), jnp.float32),
            pltpu.VMEM((L, L), jnp.float32),
            pltpu.VMEM((_SUB, 128), jnp.float32),
            pltpu.VMEM((_SUB, 128), jnp.float32),
        ],
    )(queries, keys, values, cnt)
